# baseline (speedup 1.0000x reference)
import functools

import jax
import jax.numpy as jnp
from jax import lax
from jax.experimental import pallas as pl
from jax.experimental.pallas import tpu as pltpu
from jax.experimental.pallas import tpu_sc as plsc

EMB_DIM = 32
NEG_K = 20
NUM_CORES = 2
NUM_SUBCORES = 16
NUM_WORKERS = NUM_CORES * NUM_SUBCORES
CHUNK = 32
GROUP = 16
PACK = 128 // EMB_DIM


def _sc_dots(cen_idx, ctx_idx, neg_idx_t, syn0_sr, syn1_sr):
    B = cen_idx.shape[0]
    per_w = B // NUM_WORKERS
    n_chunks = per_w // CHUNK
    out_per_chunk = CHUNK * (1 + NEG_K)
    mesh = plsc.VectorSubcoreMesh(core_axis_name="c", subcore_axis_name="s")

    @functools.partial(
        pl.kernel,
        out_type=jax.ShapeDtypeStruct((B * (1 + NEG_K),), jnp.float32),
        mesh=mesh,
        scratch_types=[
            pltpu.VMEM((CHUNK,), jnp.int32),
            pltpu.VMEM((CHUNK,), jnp.int32),
            pltpu.VMEM((NEG_K, CHUNK), jnp.int32),
            pltpu.VMEM((CHUNK,), jnp.int32),
            pltpu.VMEM((CHUNK,), jnp.int32),
            pltpu.VMEM((NEG_K, CHUNK), jnp.int32),
            pltpu.VMEM((CHUNK, 128), jnp.float32),
            pltpu.VMEM((CHUNK, 128), jnp.float32),
            pltpu.VMEM((NEG_K * CHUNK, 128), jnp.float32),
            pltpu.VMEM((CHUNK * (1 + NEG_K),), jnp.float32),
            pltpu.SemaphoreType.DMA,
        ],
        compiler_params=pltpu.CompilerParams(needs_layout_passes=False),
    )
    def sc_kernel(cen_hbm, ctx_hbm, neg_hbm, syn0_hbm, syn1_hbm, out_hbm,
                  cen_i, ctx_i, neg_i, cen_s, ctx_s, neg_s,
                  cen_r, ctx_r, neg_r, ob, sem):
        wid = lax.axis_index("s") * NUM_CORES + lax.axis_index("c")
        iota = lax.iota(jnp.int32, GROUP)
        n_groups = CHUNK // GROUP

        def chunk_body(c, carry):
            base = wid * per_w + c * CHUNK
            pltpu.sync_copy(cen_hbm.at[pl.ds(base, CHUNK)], cen_i)
            pltpu.sync_copy(ctx_hbm.at[pl.ds(base, CHUNK)], ctx_i)
            for k in range(NEG_K):
                pltpu.sync_copy(neg_hbm.at[k, pl.ds(base, CHUNK)],
                                neg_i.at[k])
            for g in range(n_groups):
                sl = pl.ds(g * GROUP, GROUP)
                cen_s[sl] = cen_i[sl] // PACK
                ctx_s[sl] = ctx_i[sl] // PACK
                for k in range(NEG_K):
                    neg_s[k, sl] = neg_i[k, sl] // PACK
            copies = [
                pltpu.async_copy(syn0_hbm.at[cen_s], cen_r, sem),
                pltpu.async_copy(syn1_hbm.at[ctx_s], ctx_r, sem),
            ]
            for k in range(NEG_K):
                copies.append(pltpu.async_copy(
                    syn1_hbm.at[neg_s.at[k]],
                    neg_r.at[pl.ds(k * CHUNK, CHUNK)], sem))
            for cp in copies:
                cp.wait()

            def group_body(g, gcarry):
                sl = pl.ds(g * GROUP, GROUP)
                e = g * GROUP + iota
                cen_off = (cen_i[sl] % PACK) * EMB_DIM
                cen_d = [plsc.load_gather(cen_r, [e, cen_off + d])
                         for d in range(EMB_DIM)]

                def dot_against(rows_ref, row_idx, off):
                    p = [cen_d[d] * plsc.load_gather(rows_ref,
                                                     [row_idx, off + d])
                         for d in range(EMB_DIM)]
                    while len(p) > 1:
                        p = [p[i] + p[i + 1] for i in range(0, len(p), 2)]
                    return p[0]

                ctx_off = (ctx_i[sl] % PACK) * EMB_DIM
                ob[sl] = dot_against(ctx_r, e, ctx_off)
                unroll = 4

                def neg_body(kq, kcarry):
                    kk0 = kq * unroll
                    for u in range(unroll):
                        kk = kk0 + u
                        n_off = (neg_i[kk, sl] % PACK) * EMB_DIM
                        acc = dot_against(neg_r, e + kk * CHUNK, n_off)
                        ob[pl.ds(CHUNK + kk * CHUNK + g * GROUP,
                                 GROUP)] = -acc
                    return kcarry

                lax.fori_loop(0, NEG_K // unroll, neg_body, 0)
                return gcarry

            lax.fori_loop(0, n_groups, group_body, 0)
            pltpu.sync_copy(
                ob,
                out_hbm.at[pl.ds((wid * n_chunks + c) * out_per_chunk,
                                 out_per_chunk)])
            return carry

        lax.fori_loop(0, n_chunks, chunk_body, 0)

    return sc_kernel(cen_idx, ctx_idx, neg_idx_t, syn0_sr, syn1_sr)


def _tc_loss(dots):
    n = dots.shape[0]
    x2 = dots.reshape(n // 128, 128)

    def body(x_ref, o_ref):
        x = x_ref[...]
        ls = jnp.minimum(x, 0.0) - jnp.log1p(jnp.exp(-jnp.abs(x)))
        o_ref[0, 0] = -jnp.sum(jnp.sum(ls, axis=1))

    out = pl.pallas_call(
        body,
        out_shape=jax.ShapeDtypeStruct((1, 1), jnp.float32),
        out_specs=pl.BlockSpec(memory_space=pltpu.SMEM),
    )(x2)
    return out[0, 0]


def kernel(center_word, context_word, neg_sampling_words, syn0, syn1):
    cen = center_word.astype(jnp.int32)
    ctx = context_word.astype(jnp.int32)
    neg_t = neg_sampling_words.astype(jnp.int32).T
    v = syn0.shape[0]
    syn0_sr = syn0.reshape(v // PACK, 128)
    syn1_sr = syn1.reshape(v // PACK, 128)
    dots = _sc_dots(cen, ctx, neg_t, syn0_sr, syn1_sr)
    return _tc_loss(dots)
